# baseline (speedup 1.0000x reference)
import functools

import jax
import jax.numpy as jnp
from jax.experimental import pallas as pl
from jax.experimental.pallas import tpu as pltpu

_CP = pltpu.CompilerParams(
    vmem_limit_bytes=134217728,
    dimension_semantics=("parallel",),
)
_BF = jnp.bfloat16
_F32 = jnp.float32


def _f32(x):
    return x.astype(_F32)


def _diag_correction(L_ref, bm):
    w = ((bm + 127) // 128 + 1) * 128
    gbase = pl.program_id(0) * bm
    s = (gbase // 128) * 128
    off = gbase - s
    sub = L_ref[:, pl.ds(s, w)]
    cols = jax.lax.broadcasted_iota(jnp.int32, (bm, w), 1)
    rows = jax.lax.broadcasted_iota(jnp.int32, (bm, w), 0)
    ldiag = jnp.sum(jnp.where(cols == rows + off, sub, 0.0), axis=1,
                    keepdims=True)
    ln_d = 2.0 * ldiag - 1.0
    return _f32(ln_d.astype(_BF)) - 2.0 * _f32(ldiag.astype(_BF))


def _step1_kernel(bm, L_ref, Hbf_ref, Hbr_ref, t1_ref, t1b_ref, lb_ref,
                  c_ref):
    lb = L_ref[...].astype(_BF)
    lb_ref[...] = lb
    c = _diag_correction(L_ref, bm)
    c_ref[...] = c
    p = jnp.dot(lb, Hbf_ref[...], preferred_element_type=_F32)
    t1 = 2.0 * p + c * _f32(Hbr_ref[...])
    t1_ref[...] = t1
    t1b_ref[...] = t1.astype(_BF)


def _step23_kernel(bm2, Lb_ref, T1bf_ref, Hbf_ref, Hr_ref, T1r_ref, c_ref,
                   Wb_ref, b_ref, out_ref, t2s_ref):
    p = pl.program_id(0)
    i = pl.program_id(1)
    rows = pl.ds(i * bm2, bm2)

    @pl.when(p == 0)
    def _phase2():
        pm = jnp.dot(Lb_ref[...], T1bf_ref[...], preferred_element_type=_F32)
        t2 = (4.0 * pm + 2.0 * c_ref[...] * _f32(T1bf_ref[rows, :])
              - Hr_ref[...])
        t2s_ref[rows, :] = t2.astype(_BF)
        out_ref[...] = jnp.zeros_like(out_ref)

    @pl.when(p == 1)
    def _phase3():
        pm = jnp.dot(Lb_ref[...], t2s_ref[...], preferred_element_type=_F32)
        t2b = _f32(t2s_ref[rows, :])
        t3 = 4.0 * pm + 2.0 * c_ref[...] * t2b - T1r_ref[...]
        s = (_f32(Hbf_ref[rows, :]) + _f32(T1bf_ref[rows, :]) + t2b
             + _f32(t3.astype(_BF)))
        out_ref[...] = (
            jnp.dot(s.astype(_BF), Wb_ref[...], preferred_element_type=_F32)
            + b_ref[...]
        )


@functools.partial(jax.jit, static_argnames=("bm", "bm2"))
def _cheb(structure, H, W, bias, bm, bm2):
    n, d = H.shape
    d_out = W.shape[1]
    grid = (n // bm,)
    grid2 = (n // bm2,)
    l_spec = pl.BlockSpec((bm, n), lambda i: (i, 0))
    l2_spec = pl.BlockSpec((bm2, n), lambda i: (i, 0))
    full_spec = pl.BlockSpec((n, d), lambda i: (0, 0))
    row_spec = pl.BlockSpec((bm, d), lambda i: (i, 0))
    row2_spec = pl.BlockSpec((bm2, d), lambda i: (i, 0))
    c_spec = pl.BlockSpec((bm, 1), lambda i: (i, 0))
    c2_spec = pl.BlockSpec((bm2, 1), lambda i: (i, 0))
    w_spec = pl.BlockSpec((d, d_out), lambda i: (0, 0))
    b_spec = pl.BlockSpec((1, d_out), lambda i: (0, 0))
    out_row_spec = pl.BlockSpec((bm2, d_out), lambda i: (i, 0))

    Hb = H.astype(_BF)
    Wb = W.astype(_BF)
    b2 = bias.reshape(1, d_out)
    rowF = jax.ShapeDtypeStruct((n, d), _F32)
    rowB = jax.ShapeDtypeStruct((n, d), _BF)

    t1, t1b, lbf, c = pl.pallas_call(
        functools.partial(_step1_kernel, bm),
        grid=grid,
        in_specs=[l_spec, full_spec, row_spec],
        out_specs=(row_spec, row_spec, l_spec, c_spec),
        out_shape=(rowF, rowB, jax.ShapeDtypeStruct((n, n), _BF),
                   jax.ShapeDtypeStruct((n, 1), _F32)),
        compiler_params=_CP,
    )(structure, Hb, Hb)

    out = pl.pallas_call(
        functools.partial(_step23_kernel, bm2),
        grid=(2, n // bm2),
        in_specs=[
            pl.BlockSpec((bm2, n), lambda p, i: (i, 0)),
            pl.BlockSpec((n, d), lambda p, i: (0, 0)),
            pl.BlockSpec((n, d), lambda p, i: (0, 0)),
            pl.BlockSpec((bm2, d), lambda p, i: (i, 0)),
            pl.BlockSpec((bm2, d), lambda p, i: (i, 0)),
            pl.BlockSpec((bm2, 1), lambda p, i: (i, 0)),
            pl.BlockSpec((d, d_out), lambda p, i: (0, 0)),
            pl.BlockSpec((1, d_out), lambda p, i: (0, 0)),
        ],
        out_specs=pl.BlockSpec((bm2, d_out), lambda p, i: (i, 0)),
        out_shape=jax.ShapeDtypeStruct((n, d_out), _F32),
        scratch_shapes=[pltpu.VMEM((n, d), _BF)],
        compiler_params=pltpu.CompilerParams(
            vmem_limit_bytes=134217728,
            dimension_semantics=("arbitrary", "arbitrary"),
        ),
    )(lbf, t1b, Hb, H, t1, c, Wb, b2)
    return out


def kernel(structure, H, W, bias):
    n = structure.shape[0]
    bm = 200 if n % 400 == 0 else 8
    bm2 = 1000 if n % 1000 == 0 else bm
    return _cheb(structure, H, W, bias, bm, bm2)


# speedup vs baseline: 1.0057x; 1.0057x over previous
import functools

import jax
import jax.numpy as jnp
from jax.experimental import pallas as pl
from jax.experimental.pallas import tpu as pltpu

_CP = pltpu.CompilerParams(
    vmem_limit_bytes=134217728,
    dimension_semantics=("parallel",),
)
_BF = jnp.bfloat16
_F32 = jnp.float32


def _f32(x):
    return x.astype(_F32)


def _diag_correction(L_ref, bm):
    w = ((bm + 127) // 128 + 1) * 128
    gbase = pl.program_id(0) * bm
    s = (gbase // 128) * 128
    off = gbase - s
    sub = L_ref[:, pl.ds(s, w)]
    cols = jax.lax.broadcasted_iota(jnp.int32, (bm, w), 1)
    rows = jax.lax.broadcasted_iota(jnp.int32, (bm, w), 0)
    ldiag = jnp.sum(jnp.where(cols == rows + off, sub, 0.0), axis=1,
                    keepdims=True)
    ln_d = 2.0 * ldiag - 1.0
    return _f32(ln_d.astype(_BF)) - 2.0 * _f32(ldiag.astype(_BF))


def _step1_kernel(bm, L_ref, Hbf_ref, Hbr_ref, t1_ref, t1b_ref, lb_ref,
                  c_ref):
    lb = L_ref[...].astype(_BF)
    lb_ref[...] = lb
    c = _diag_correction(L_ref, bm)
    c_ref[...] = c
    p = jnp.dot(lb, Hbf_ref[...], preferred_element_type=_F32)
    t1 = 2.0 * p + c * _f32(Hbr_ref[...])
    t1_ref[...] = t1
    t1b_ref[...] = t1.astype(_BF)


def _step23_kernel(bm2, Lb_ref, T1bf_ref, Hbf_ref, Hr_ref, T1r_ref, c_ref,
                   Wb_ref, b_ref, out_ref, t2s_ref):
    p = pl.program_id(0)
    i = pl.program_id(1)
    rows = pl.ds(i * bm2, bm2)

    @pl.when(p == 0)
    def _phase2():
        pm = jnp.dot(Lb_ref[...], T1bf_ref[...], preferred_element_type=_F32)
        t2 = (4.0 * pm + 2.0 * c_ref[...] * _f32(T1bf_ref[rows, :])
              - Hr_ref[...])
        t2s_ref[rows, :] = t2.astype(_BF)

    @pl.when(p == 1)
    def _phase3():
        pm = jnp.dot(Lb_ref[...], t2s_ref[...], preferred_element_type=_F32)
        t2b = _f32(t2s_ref[rows, :])
        t3 = 4.0 * pm + 2.0 * c_ref[...] * t2b - T1r_ref[...]
        s = (_f32(Hbf_ref[rows, :]) + _f32(T1bf_ref[rows, :]) + t2b
             + _f32(t3.astype(_BF)))
        out_ref[...] = (
            jnp.dot(s.astype(_BF), Wb_ref[...], preferred_element_type=_F32)
            + b_ref[...]
        )


@functools.partial(jax.jit, static_argnames=("bm", "bm2"))
def _cheb(structure, H, W, bias, bm, bm2):
    n, d = H.shape
    d_out = W.shape[1]
    grid = (n // bm,)
    grid2 = (n // bm2,)
    l_spec = pl.BlockSpec((bm, n), lambda i: (i, 0))
    l2_spec = pl.BlockSpec((bm2, n), lambda i: (i, 0))
    full_spec = pl.BlockSpec((n, d), lambda i: (0, 0))
    row_spec = pl.BlockSpec((bm, d), lambda i: (i, 0))
    row2_spec = pl.BlockSpec((bm2, d), lambda i: (i, 0))
    c_spec = pl.BlockSpec((bm, 1), lambda i: (i, 0))
    c2_spec = pl.BlockSpec((bm2, 1), lambda i: (i, 0))
    w_spec = pl.BlockSpec((d, d_out), lambda i: (0, 0))
    b_spec = pl.BlockSpec((1, d_out), lambda i: (0, 0))
    out_row_spec = pl.BlockSpec((bm2, d_out), lambda i: (i, 0))

    Hb = H.astype(_BF)
    Wb = W.astype(_BF)
    b2 = bias.reshape(1, d_out)
    rowF = jax.ShapeDtypeStruct((n, d), _F32)
    rowB = jax.ShapeDtypeStruct((n, d), _BF)

    t1, t1b, lbf, c = pl.pallas_call(
        functools.partial(_step1_kernel, bm),
        grid=grid,
        in_specs=[l_spec, full_spec, row_spec],
        out_specs=(row_spec, row_spec, l_spec, c_spec),
        out_shape=(rowF, rowB, jax.ShapeDtypeStruct((n, n), _BF),
                   jax.ShapeDtypeStruct((n, 1), _F32)),
        compiler_params=_CP,
    )(structure, Hb, Hb)

    out = pl.pallas_call(
        functools.partial(_step23_kernel, bm2),
        grid=(2, n // bm2),
        in_specs=[
            pl.BlockSpec((bm2, n), lambda p, i: (i, 0)),
            pl.BlockSpec((n, d), lambda p, i: (0, 0)),
            pl.BlockSpec((n, d), lambda p, i: (0, 0)),
            pl.BlockSpec((bm2, d),
                         lambda p, i: (jax.lax.select(p == 0, i, 0), 0)),
            pl.BlockSpec((bm2, d),
                         lambda p, i: (jax.lax.select(p == 0, 0, i), 0)),
            pl.BlockSpec((bm2, 1), lambda p, i: (i, 0)),
            pl.BlockSpec((d, d_out), lambda p, i: (0, 0)),
            pl.BlockSpec((1, d_out), lambda p, i: (0, 0)),
        ],
        out_specs=pl.BlockSpec(
            (bm2, d_out), lambda p, i: (jax.lax.select(p == 0, 0, i), 0)),
        out_shape=jax.ShapeDtypeStruct((n, d_out), _F32),
        scratch_shapes=[pltpu.VMEM((n, d), _BF)],
        compiler_params=pltpu.CompilerParams(
            vmem_limit_bytes=134217728,
            dimension_semantics=("arbitrary", "arbitrary"),
        ),
    )(lbf, t1b, Hb, H, t1, c, Wb, b2)
    return out


def kernel(structure, H, W, bias):
    n = structure.shape[0]
    bm = 200 if n % 400 == 0 else 8
    bm2 = 1000 if n % 1000 == 0 else bm
    return _cheb(structure, H, W, bias, bm, bm2)


# speedup vs baseline: 1.0100x; 1.0043x over previous
import functools

import jax
import jax.numpy as jnp
from jax.experimental import pallas as pl
from jax.experimental.pallas import tpu as pltpu

_CP = pltpu.CompilerParams(
    vmem_limit_bytes=134217728,
    dimension_semantics=("parallel",),
)
_BF = jnp.bfloat16
_F32 = jnp.float32


def _f32(x):
    return x.astype(_F32)


def _diag_correction(L_ref, bm):
    w = ((bm + 127) // 128 + 1) * 128
    gbase = pl.program_id(0) * bm
    s = (gbase // 128) * 128
    off = gbase - s
    sub = L_ref[:, pl.ds(s, w)]
    cols = jax.lax.broadcasted_iota(jnp.int32, (bm, w), 1)
    rows = jax.lax.broadcasted_iota(jnp.int32, (bm, w), 0)
    ldiag = jnp.sum(jnp.where(cols == rows + off, sub, 0.0), axis=1,
                    keepdims=True)
    ln_d = 2.0 * ldiag - 1.0
    return _f32(ln_d.astype(_BF)) - 2.0 * _f32(ldiag.astype(_BF))


def _step1_kernel(bm, L_ref, Hbf_ref, Hbr_ref, t1_ref, t1b_ref, lb_ref,
                  c_ref):
    lb = L_ref[...].astype(_BF)
    lb_ref[...] = lb
    c = _diag_correction(L_ref, bm)
    c_ref[...] = c
    p = jnp.dot(lb, Hbf_ref[...], preferred_element_type=_F32)
    t1 = 2.0 * p + c * _f32(Hbr_ref[...])
    t1_ref[...] = t1
    t1b_ref[...] = t1.astype(_BF)


def _step23_kernel(bm2, Lb_ref, T1bf_ref, Hbf_ref, Hr_ref, T1r_ref, c_ref,
                   Wb_ref, b_ref, out_ref, t2s_ref):
    p = pl.program_id(0)
    i = pl.program_id(1)
    rows = pl.ds(i * bm2, bm2)

    @pl.when(p == 0)
    def _phase2():
        pm = jnp.dot(Lb_ref[...], T1bf_ref[...], preferred_element_type=_F32)
        t2 = (4.0 * pm + 2.0 * c_ref[...] * _f32(T1bf_ref[rows, :])
              - Hr_ref[...])
        t2s_ref[rows, :] = t2.astype(_BF)

    @pl.when(p == 1)
    def _phase3():
        pm = jnp.dot(Lb_ref[...], t2s_ref[...], preferred_element_type=_F32)
        t2b = _f32(t2s_ref[rows, :])
        t3 = 4.0 * pm + 2.0 * c_ref[...] * t2b - T1r_ref[...]
        s = (_f32(Hbf_ref[rows, :]) + _f32(T1bf_ref[rows, :]) + t2b
             + _f32(t3.astype(_BF)))
        out_ref[...] = (
            jnp.dot(s.astype(_BF), Wb_ref[...], preferred_element_type=_F32)
            + b_ref[...]
        )


@functools.partial(jax.jit, static_argnames=("bm", "bm2"))
def _cheb(structure, H, W, bias, bm, bm2):
    n, d = H.shape
    d_out = W.shape[1]
    grid = (n // bm,)
    grid2 = (n // bm2,)
    l_spec = pl.BlockSpec((bm, n), lambda i: (i, 0))
    l2_spec = pl.BlockSpec((bm2, n), lambda i: (i, 0))
    full_spec = pl.BlockSpec((n, d), lambda i: (0, 0))
    row_spec = pl.BlockSpec((bm, d), lambda i: (i, 0))
    row2_spec = pl.BlockSpec((bm2, d), lambda i: (i, 0))
    c_spec = pl.BlockSpec((bm, 1), lambda i: (i, 0))
    c2_spec = pl.BlockSpec((bm2, 1), lambda i: (i, 0))
    w_spec = pl.BlockSpec((d, d_out), lambda i: (0, 0))
    b_spec = pl.BlockSpec((1, d_out), lambda i: (0, 0))
    out_row_spec = pl.BlockSpec((bm2, d_out), lambda i: (i, 0))

    Hb = H.astype(_BF)
    Wb = W.astype(_BF)
    b2 = bias.reshape(1, d_out)
    rowF = jax.ShapeDtypeStruct((n, d), _F32)
    rowB = jax.ShapeDtypeStruct((n, d), _BF)

    t1, t1b, lbf, c = pl.pallas_call(
        functools.partial(_step1_kernel, bm),
        grid=grid,
        in_specs=[l_spec, full_spec, row_spec],
        out_specs=(row_spec, row_spec, l_spec, c_spec),
        out_shape=(rowF, rowB, jax.ShapeDtypeStruct((n, n), _BF),
                   jax.ShapeDtypeStruct((n, 1), _F32)),
        compiler_params=_CP,
    )(structure, Hb, Hb)

    out = pl.pallas_call(
        functools.partial(_step23_kernel, bm2),
        grid=(2, n // bm2),
        in_specs=[
            pl.BlockSpec((bm2, n), lambda p, i: (i, 0)),
            pl.BlockSpec((n, d), lambda p, i: (0, 0)),
            pl.BlockSpec((n, d), lambda p, i: (0, 0)),
            pl.BlockSpec((bm2, d),
                         lambda p, i: (jax.lax.select(p == 0, i, 0), 0)),
            pl.BlockSpec((bm2, d),
                         lambda p, i: (jax.lax.select(p == 0, 0, i), 0)),
            pl.BlockSpec((bm2, 1), lambda p, i: (i, 0)),
            pl.BlockSpec((d, d_out), lambda p, i: (0, 0)),
            pl.BlockSpec((1, d_out), lambda p, i: (0, 0)),
        ],
        out_specs=pl.BlockSpec(
            (bm2, d_out), lambda p, i: (jax.lax.select(p == 0, 0, i), 0)),
        out_shape=jax.ShapeDtypeStruct((n, d_out), _F32),
        scratch_shapes=[pltpu.VMEM((n, d), _BF)],
        compiler_params=pltpu.CompilerParams(
            vmem_limit_bytes=134217728,
            dimension_semantics=("arbitrary", "arbitrary"),
        ),
    )(lbf, t1b, Hb, H, t1, c, Wb, b2)
    return out


def kernel(structure, H, W, bias):
    n = structure.shape[0]
    bm = 400 if n % 400 == 0 else 8
    bm2 = 1000 if n % 1000 == 0 else bm
    return _cheb(structure, H, W, bias, bm, bm2)


# speedup vs baseline: 1.0107x; 1.0006x over previous
import functools

import jax
import jax.numpy as jnp
from jax.experimental import pallas as pl
from jax.experimental.pallas import tpu as pltpu

_CP = pltpu.CompilerParams(
    vmem_limit_bytes=134217728,
    dimension_semantics=("parallel",),
)
_BF = jnp.bfloat16
_F32 = jnp.float32


def _f32(x):
    return x.astype(_F32)


def _diag_correction(L_ref, bm):
    w = ((bm + 127) // 128 + 1) * 128
    gbase = pl.program_id(0) * bm
    s = (gbase // 128) * 128
    off = gbase - s
    sub = L_ref[:, pl.ds(s, w)]
    cols = jax.lax.broadcasted_iota(jnp.int32, (bm, w), 1)
    rows = jax.lax.broadcasted_iota(jnp.int32, (bm, w), 0)
    ldiag = jnp.sum(jnp.where(cols == rows + off, sub, 0.0), axis=1,
                    keepdims=True)
    ln_d = 2.0 * ldiag - 1.0
    return _f32(ln_d.astype(_BF)) - 2.0 * _f32(ldiag.astype(_BF))


def _step1_kernel(bm, L_ref, Hbf_ref, Hbr_ref, t1_ref, t1b_ref, lb_ref,
                  c_ref):
    lb = L_ref[...].astype(_BF)
    lb_ref[...] = lb
    c = _diag_correction(L_ref, bm)
    c_ref[...] = c
    p = jnp.dot(lb, Hbf_ref[...], preferred_element_type=_F32)
    t1 = 2.0 * p + c * _f32(Hbr_ref[...])
    t1_ref[...] = t1
    t1b_ref[...] = t1.astype(_BF)


def _step23_kernel(bm2, g1, Lb_ref, T1bf_ref, Hbf_ref, Hr_ref, T1r_ref,
                   c_ref, Wb_ref, b_ref, out_ref, t2s_ref):
    p = pl.program_id(0)
    i = pl.program_id(1)
    r = jax.lax.select(p == 0, i, g1 - i)
    rows = pl.ds(r * bm2, bm2)

    @pl.when(p == 0)
    def _phase2():
        pm = jnp.dot(Lb_ref[...], T1bf_ref[...], preferred_element_type=_F32)
        t2 = (4.0 * pm + 2.0 * c_ref[...] * _f32(T1bf_ref[rows, :])
              - Hr_ref[...])
        t2s_ref[rows, :] = t2.astype(_BF)

    @pl.when(p == 1)
    def _phase3():
        pm = jnp.dot(Lb_ref[...], t2s_ref[...], preferred_element_type=_F32)
        t2b = _f32(t2s_ref[rows, :])
        t3 = 4.0 * pm + 2.0 * c_ref[...] * t2b - T1r_ref[...]
        s = (_f32(Hbf_ref[rows, :]) + _f32(T1bf_ref[rows, :]) + t2b
             + _f32(t3.astype(_BF)))
        out_ref[...] = (
            jnp.dot(s.astype(_BF), Wb_ref[...], preferred_element_type=_F32)
            + b_ref[...]
        )


@functools.partial(jax.jit, static_argnames=("bm", "bm2"))
def _cheb(structure, H, W, bias, bm, bm2):
    n, d = H.shape
    d_out = W.shape[1]
    grid = (n // bm,)
    grid2 = (n // bm2,)
    l_spec = pl.BlockSpec((bm, n), lambda i: (i, 0))
    l2_spec = pl.BlockSpec((bm2, n), lambda i: (i, 0))
    full_spec = pl.BlockSpec((n, d), lambda i: (0, 0))
    row_spec = pl.BlockSpec((bm, d), lambda i: (i, 0))
    row2_spec = pl.BlockSpec((bm2, d), lambda i: (i, 0))
    c_spec = pl.BlockSpec((bm, 1), lambda i: (i, 0))
    c2_spec = pl.BlockSpec((bm2, 1), lambda i: (i, 0))
    w_spec = pl.BlockSpec((d, d_out), lambda i: (0, 0))
    b_spec = pl.BlockSpec((1, d_out), lambda i: (0, 0))
    out_row_spec = pl.BlockSpec((bm2, d_out), lambda i: (i, 0))

    Hb = H.astype(_BF)
    Wb = W.astype(_BF)
    b2 = bias.reshape(1, d_out)
    rowF = jax.ShapeDtypeStruct((n, d), _F32)
    rowB = jax.ShapeDtypeStruct((n, d), _BF)

    t1, t1b, lbf, c = pl.pallas_call(
        functools.partial(_step1_kernel, bm),
        grid=grid,
        in_specs=[l_spec, full_spec, row_spec],
        out_specs=(row_spec, row_spec, l_spec, c_spec),
        out_shape=(rowF, rowB, jax.ShapeDtypeStruct((n, n), _BF),
                   jax.ShapeDtypeStruct((n, 1), _F32)),
        compiler_params=_CP,
    )(structure, Hb, Hb)

    g1 = n // bm2 - 1
    out = pl.pallas_call(
        functools.partial(_step23_kernel, bm2, g1),
        grid=(2, n // bm2),
        in_specs=[
            pl.BlockSpec((bm2, n),
                         lambda p, i: (jax.lax.select(p == 0, i, g1 - i), 0)),
            pl.BlockSpec((n, d), lambda p, i: (0, 0)),
            pl.BlockSpec((n, d), lambda p, i: (0, 0)),
            pl.BlockSpec((bm2, d),
                         lambda p, i: (jax.lax.select(p == 0, i, g1), 0)),
            pl.BlockSpec((bm2, d),
                         lambda p, i: (jax.lax.select(p == 0, g1, g1 - i), 0)),
            pl.BlockSpec((bm2, 1),
                         lambda p, i: (jax.lax.select(p == 0, i, g1 - i), 0)),
            pl.BlockSpec((d, d_out), lambda p, i: (0, 0)),
            pl.BlockSpec((1, d_out), lambda p, i: (0, 0)),
        ],
        out_specs=pl.BlockSpec(
            (bm2, d_out),
            lambda p, i: (jax.lax.select(p == 0, g1, g1 - i), 0)),
        out_shape=jax.ShapeDtypeStruct((n, d_out), _F32),
        scratch_shapes=[pltpu.VMEM((n, d), _BF)],
        compiler_params=pltpu.CompilerParams(
            vmem_limit_bytes=134217728,
            dimension_semantics=("arbitrary", "arbitrary"),
        ),
    )(lbf, t1b, Hb, H, t1, c, Wb, b2)
    return out


def kernel(structure, H, W, bias):
    n = structure.shape[0]
    bm = 400 if n % 400 == 0 else 8
    bm2 = 1000 if n % 1000 == 0 else bm
    return _cheb(structure, H, W, bias, bm, bm2)
